# baseline (speedup 1.0000x reference)
import functools

import jax
import jax.numpy as jnp
from jax import lax
from jax.experimental import pallas as pl
from jax.experimental.pallas import tpu as pltpu
from jax.experimental.pallas import tpu_sc as plsc

E = 16
H = 768
I = 1024
T = 2048
B = 128
NB = T // B + E
NW = 32
TPW = T // NW
L = 16

_mesh = plsc.VectorSubcoreMesh(core_axis_name="c", subcore_axis_name="s")
_sc_params = pltpu.CompilerParams(needs_layout_passes=False)


def _wid():
    return lax.axis_index("s") * 2 + lax.axis_index("c")


def _route_body(logits_hbm, ids_hbm, phist_hbm, logits_v, ids_v, ph_v):
    w = _wid()
    t0 = w * TPW
    pltpu.sync_copy(logits_hbm.at[pl.ds(t0 * E, TPW * E)], logits_v)
    iota = lax.iota(jnp.int32, L)
    hist = jnp.zeros((L,), jnp.int32)
    for g in range(TPW // L):
        base = (g * L + iota) * E
        best_v = plsc.load_gather(logits_v, [base])
        best_i = jnp.zeros((L,), jnp.int32)
        for e in range(1, E):
            ve = plsc.load_gather(logits_v, [base + e])
            upd = ve > best_v
            best_v = jnp.where(upd, ve, best_v)
            best_i = jnp.where(upd, e, best_i)
        ids_v[pl.ds(g * L, L)] = best_i
        for e in range(E):
            cnt = plsc.all_reduce_population_count(best_i == e)
            hist = hist + jnp.where(iota == e, cnt, 0)
    ph_v[0] = hist
    pltpu.sync_copy(ids_v, ids_hbm.at[pl.ds(t0, TPW)])
    pltpu.sync_copy(ph_v, phist_hbm.at[pl.ds(w, 1)])


def _slots_body(x_hbm, ids_hbm, phist_hbm,
                slots_hbm, sortedx_hbm, be_hbm,
                ph_v, myids_v, offv_v, slots_v, xrows_v, tmp_v, bev_v, sem):
    w = _wid()
    t0 = w * TPW
    pltpu.sync_copy(phist_hbm, ph_v)
    pltpu.sync_copy(ids_hbm.at[pl.ds(t0, TPW)], myids_v)
    iota = lax.iota(jnp.int32, L)
    totals = jnp.zeros((L,), jnp.int32)
    myprefix = jnp.zeros((L,), jnp.int32)
    for ww in range(NW):
        row = ph_v[ww]
        totals = totals + row
        flag = (jnp.array(ww, jnp.int32) < w).astype(jnp.int32)
        myprefix = myprefix + row * flag
    nb = (totals + (B - 1)) // B
    baseblk = plsc.cumsum(nb) - nb
    offv_v[...] = baseblk * B + myprefix
    for g in range(TPW // L):
        ids_g = myids_v[pl.ds(g * L, L)]
        offs = plsc.load_gather(offv_v, [ids_g])
        rank = jnp.zeros((L,), jnp.int32)
        upd = jnp.zeros((L,), jnp.int32)
        for e in range(E):
            m = ids_g == e
            cs = plsc.cumsum(m.astype(jnp.int32))
            rank = rank + jnp.where(m, cs - 1, 0)
            cnt = plsc.all_reduce_population_count(m)
            upd = upd + jnp.where(iota == e, cnt, 0)
        slots_v[pl.ds(g * L, L)] = offs + rank
        offv_v[...] = offv_v[...] + upd
    pltpu.sync_copy(slots_v, slots_hbm.at[pl.ds(t0, TPW)])
    pltpu.sync_copy(x_hbm.at[pl.ds(t0, TPW)], xrows_v)
    pltpu.async_copy(xrows_v, sortedx_hbm.at[slots_v], sem).wait()

    @pl.when(w == 0)
    def _():
        tmp_v[pl.ds(0, L)] = jnp.zeros((L,), jnp.int32)
        tmp_v[pl.ds(L, L)] = jnp.zeros((L,), jnp.int32)
        plsc.store_scatter(tmp_v, [baseblk], iota, mask=nb > 0)
        lo = tmp_v[pl.ds(0, L)]
        hi = tmp_v[pl.ds(L, L)]
        locm = plsc.cummax(lo)
        m0 = jnp.max(locm)
        hicm = jnp.maximum(plsc.cummax(hi), m0)
        bev_v[pl.ds(0, L)] = locm
        bev_v[pl.ds(L, L)] = hicm
        bev_v[pl.ds(2 * L, L)] = jnp.zeros((L,), jnp.int32) + jnp.sum(nb)
        pltpu.sync_copy(bev_v, be_hbm)


def _combine_body(y_hbm, slots_hbm, out_hbm, slots_v, yrows_v, sem):
    w = _wid()
    t0 = w * TPW
    pltpu.sync_copy(slots_hbm.at[pl.ds(t0, TPW)], slots_v)
    pltpu.async_copy(y_hbm.at[slots_v], yrows_v, sem).wait()
    pltpu.sync_copy(yrows_v, out_hbm.at[pl.ds(t0, TPW)])


_route = functools.partial(
    pl.kernel,
    out_type=[jax.ShapeDtypeStruct((T,), jnp.int32),
              jax.ShapeDtypeStruct((NW, L), jnp.int32)],
    mesh=_mesh,
    scratch_types=[pltpu.VMEM((TPW * E,), jnp.float32),
                   pltpu.VMEM((TPW,), jnp.int32),
                   pltpu.VMEM((1, L), jnp.int32)],
    compiler_params=_sc_params,
)(_route_body)

_slots = functools.partial(
    pl.kernel,
    out_type=[jax.ShapeDtypeStruct((T,), jnp.int32),
              jax.ShapeDtypeStruct((NB * B, H), jnp.float32),
              jax.ShapeDtypeStruct((NB + L,), jnp.int32)],
    mesh=_mesh,
    scratch_types=[pltpu.VMEM((NW, L), jnp.int32),
                   pltpu.VMEM((TPW,), jnp.int32),
                   pltpu.VMEM((L,), jnp.int32),
                   pltpu.VMEM((TPW,), jnp.int32),
                   pltpu.VMEM((TPW, H), jnp.float32),
                   pltpu.VMEM((NB,), jnp.int32),
                   pltpu.VMEM((NB + L,), jnp.int32),
                   pltpu.SemaphoreType.DMA],
    compiler_params=_sc_params,
)(_slots_body)

_combine = functools.partial(
    pl.kernel,
    out_type=jax.ShapeDtypeStruct((T, H), jnp.float32),
    mesh=_mesh,
    scratch_types=[pltpu.VMEM((TPW,), jnp.int32),
                   pltpu.VMEM((TPW, H), jnp.float32),
                   pltpu.SemaphoreType.DMA],
    compiler_params=_sc_params,
)(_combine_body)


def _ffn_body(be_ref, x_ref, w1_ref, w3_ref, w2_ref, y_ref):
    @pl.when(pl.program_id(0) < be_ref[NB])
    def _():
        x = x_ref[...].astype(jnp.bfloat16)
        dn = (((1,), (1,)), ((), ()))
        g = lax.dot_general(x, w1_ref[0].astype(jnp.bfloat16), dn,
                            preferred_element_type=jnp.float32)
        u = lax.dot_general(x, w3_ref[0].astype(jnp.bfloat16), dn,
                            preferred_element_type=jnp.float32)
        a = (g * jax.nn.sigmoid(g) * u).astype(jnp.bfloat16)
        y_ref[...] = lax.dot_general(a, w2_ref[0].astype(jnp.bfloat16), dn,
                                     preferred_element_type=jnp.float32)


def _grouped_ffn(block_expert, sorted_x, w1, w2, w3):
    def xy_idx(b, be):
        return (jnp.minimum(b, be[NB] - 1), 0)

    grid_spec = pltpu.PrefetchScalarGridSpec(
        num_scalar_prefetch=1,
        grid=(NB,),
        in_specs=[
            pl.BlockSpec((B, H), xy_idx),
            pl.BlockSpec((1, I, H), lambda b, be: (be[b], 0, 0)),
            pl.BlockSpec((1, I, H), lambda b, be: (be[b], 0, 0)),
            pl.BlockSpec((1, H, I), lambda b, be: (be[b], 0, 0)),
        ],
        out_specs=pl.BlockSpec((B, H), xy_idx),
    )
    return pl.pallas_call(
        _ffn_body,
        grid_spec=grid_spec,
        out_shape=jax.ShapeDtypeStruct((NB * B, H), jnp.float32),
    )(block_expert, sorted_x, w1, w3, w2)


def kernel(hidden_states, router_logits, w1, w2, w3):
    ids, phist = _route(router_logits.reshape(-1))
    slots, sorted_x, block_expert = _slots(hidden_states, ids, phist)
    y = _grouped_ffn(block_expert, sorted_x, w1, w2, w3)
    return _combine(y, slots)


# speedup vs baseline: 1.1622x; 1.1622x over previous
import functools

import jax
import jax.numpy as jnp
from jax import lax
from jax.experimental import pallas as pl
from jax.experimental.pallas import tpu as pltpu
from jax.experimental.pallas import tpu_sc as plsc

E = 16
H = 768
I = 1024
T = 2048
B = 256
NB = T // B + E
NBE = 3 * 16
NW = 32
TPW = T // NW
L = 16

_mesh = plsc.VectorSubcoreMesh(core_axis_name="c", subcore_axis_name="s")
_sc_params = pltpu.CompilerParams(needs_layout_passes=False)


def _wid():
    return lax.axis_index("s") * 2 + lax.axis_index("c")


def _route_body(logits_hbm, ids_hbm, phist_hbm, logits_v, ids_v, ph_v):
    w = _wid()
    t0 = w * TPW
    pltpu.sync_copy(logits_hbm.at[pl.ds(t0 * E, TPW * E)], logits_v)
    iota = lax.iota(jnp.int32, L)
    hist = jnp.zeros((L,), jnp.int32)
    for g in range(TPW // L):
        base = (g * L + iota) * E
        best_v = plsc.load_gather(logits_v, [base])
        best_i = jnp.zeros((L,), jnp.int32)
        for e in range(1, E):
            ve = plsc.load_gather(logits_v, [base + e])
            upd = ve > best_v
            best_v = jnp.where(upd, ve, best_v)
            best_i = jnp.where(upd, e, best_i)
        ids_v[pl.ds(g * L, L)] = best_i
        for e in range(E):
            cnt = plsc.all_reduce_population_count(best_i == e)
            hist = hist + jnp.where(iota == e, cnt, 0)
    ph_v[0] = hist
    pltpu.sync_copy(ids_v, ids_hbm.at[pl.ds(t0, TPW)])
    pltpu.sync_copy(ph_v, phist_hbm.at[pl.ds(w, 1)])


def _slots_body(x_hbm, ids_hbm, phist_hbm,
                slots_hbm, sortedx_hbm, be_hbm,
                ph_v, myids_v, offv_v, slots_v, xrows_v, tmp_v, bev_v, sem):
    w = _wid()
    t0 = w * TPW
    pltpu.sync_copy(phist_hbm, ph_v)
    pltpu.sync_copy(ids_hbm.at[pl.ds(t0, TPW)], myids_v)
    iota = lax.iota(jnp.int32, L)
    totals = jnp.zeros((L,), jnp.int32)
    myprefix = jnp.zeros((L,), jnp.int32)
    for ww in range(NW):
        row = ph_v[ww]
        totals = totals + row
        flag = (jnp.array(ww, jnp.int32) < w).astype(jnp.int32)
        myprefix = myprefix + row * flag
    nb = (totals + (B - 1)) // B
    baseblk = plsc.cumsum(nb) - nb
    offv_v[...] = baseblk * B + myprefix
    for g in range(TPW // L):
        ids_g = myids_v[pl.ds(g * L, L)]
        offs = plsc.load_gather(offv_v, [ids_g])
        rank = jnp.zeros((L,), jnp.int32)
        upd = jnp.zeros((L,), jnp.int32)
        for e in range(E):
            m = ids_g == e
            cs = plsc.cumsum(m.astype(jnp.int32))
            rank = rank + jnp.where(m, cs - 1, 0)
            cnt = plsc.all_reduce_population_count(m)
            upd = upd + jnp.where(iota == e, cnt, 0)
        slots_v[pl.ds(g * L, L)] = offs + rank
        offv_v[...] = offv_v[...] + upd
    pltpu.sync_copy(slots_v, slots_hbm.at[pl.ds(t0, TPW)])
    pltpu.sync_copy(x_hbm.at[pl.ds(t0, TPW)], xrows_v)
    pltpu.async_copy(xrows_v, sortedx_hbm.at[slots_v], sem).wait()

    @pl.when(w == 0)
    def _():
        tmp_v[pl.ds(0, L)] = jnp.zeros((L,), jnp.int32)
        tmp_v[pl.ds(L, L)] = jnp.zeros((L,), jnp.int32)
        plsc.store_scatter(tmp_v, [baseblk], iota, mask=nb > 0)
        lo = tmp_v[pl.ds(0, L)]
        hi = tmp_v[pl.ds(L, L)]
        locm = plsc.cummax(lo)
        m0 = jnp.max(locm)
        hicm = jnp.maximum(plsc.cummax(hi), m0)
        bev_v[pl.ds(0, L)] = locm
        bev_v[pl.ds(L, L)] = hicm
        bev_v[pl.ds(2 * L, L)] = jnp.zeros((L,), jnp.int32) + jnp.sum(nb)
        pltpu.sync_copy(bev_v, be_hbm)


def _combine_body(y_hbm, slots_hbm, out_hbm, slots_v, yrows_v, sem):
    w = _wid()
    t0 = w * TPW
    pltpu.sync_copy(slots_hbm.at[pl.ds(t0, TPW)], slots_v)
    pltpu.async_copy(y_hbm.at[slots_v], yrows_v, sem).wait()
    pltpu.sync_copy(yrows_v, out_hbm.at[pl.ds(t0, TPW)])


_route = functools.partial(
    pl.kernel,
    out_type=[jax.ShapeDtypeStruct((T,), jnp.int32),
              jax.ShapeDtypeStruct((NW, L), jnp.int32)],
    mesh=_mesh,
    scratch_types=[pltpu.VMEM((TPW * E,), jnp.float32),
                   pltpu.VMEM((TPW,), jnp.int32),
                   pltpu.VMEM((1, L), jnp.int32)],
    compiler_params=_sc_params,
)(_route_body)

_slots = functools.partial(
    pl.kernel,
    out_type=[jax.ShapeDtypeStruct((T,), jnp.int32),
              jax.ShapeDtypeStruct((NB * B, H), jnp.float32),
              jax.ShapeDtypeStruct((NBE,), jnp.int32)],
    mesh=_mesh,
    scratch_types=[pltpu.VMEM((NW, L), jnp.int32),
                   pltpu.VMEM((TPW,), jnp.int32),
                   pltpu.VMEM((L,), jnp.int32),
                   pltpu.VMEM((TPW,), jnp.int32),
                   pltpu.VMEM((TPW, H), jnp.float32),
                   pltpu.VMEM((2 * L,), jnp.int32),
                   pltpu.VMEM((NBE,), jnp.int32),
                   pltpu.SemaphoreType.DMA],
    compiler_params=_sc_params,
)(_slots_body)

_combine = functools.partial(
    pl.kernel,
    out_type=jax.ShapeDtypeStruct((T, H), jnp.float32),
    mesh=_mesh,
    scratch_types=[pltpu.VMEM((TPW,), jnp.int32),
                   pltpu.VMEM((TPW, H), jnp.float32),
                   pltpu.SemaphoreType.DMA],
    compiler_params=_sc_params,
)(_combine_body)


def _ffn_body(be_ref, x_ref, w1_ref, w3_ref, w2_ref, y_ref):
    @pl.when(pl.program_id(0) < be_ref[2 * L])
    def _():
        x = x_ref[...].astype(jnp.bfloat16)
        dn = (((1,), (1,)), ((), ()))
        g = lax.dot_general(x, w1_ref[0].astype(jnp.bfloat16), dn,
                            preferred_element_type=jnp.float32)
        u = lax.dot_general(x, w3_ref[0].astype(jnp.bfloat16), dn,
                            preferred_element_type=jnp.float32)
        a = (g * jax.nn.sigmoid(g) * u).astype(jnp.bfloat16)
        y_ref[...] = lax.dot_general(a, w2_ref[0].astype(jnp.bfloat16), dn,
                                     preferred_element_type=jnp.float32)


def _grouped_ffn(block_expert, sorted_x, w1, w2, w3):
    def xy_idx(b, be):
        return (jnp.minimum(b, be[2 * L] - 1), 0)

    grid_spec = pltpu.PrefetchScalarGridSpec(
        num_scalar_prefetch=1,
        grid=(NB,),
        in_specs=[
            pl.BlockSpec((B, H), xy_idx),
            pl.BlockSpec((1, I, H), lambda b, be: (be[b], 0, 0)),
            pl.BlockSpec((1, I, H), lambda b, be: (be[b], 0, 0)),
            pl.BlockSpec((1, H, I), lambda b, be: (be[b], 0, 0)),
        ],
        out_specs=pl.BlockSpec((B, H), xy_idx),
    )
    return pl.pallas_call(
        _ffn_body,
        grid_spec=grid_spec,
        out_shape=jax.ShapeDtypeStruct((NB * B, H), jnp.float32),
    )(block_expert, sorted_x, w1, w3, w2)


def kernel(hidden_states, router_logits, w1, w2, w3):
    ids, phist = _route(router_logits.reshape(-1))
    slots, sorted_x, block_expert = _slots(hidden_states, ids, phist)
    y = _grouped_ffn(block_expert, sorted_x, w1, w2, w3)
    return _combine(y, slots)
